# Initial kernel scaffold; baseline (speedup 1.0000x reference)
import jax
import jax.numpy as jnp
from jax.experimental import pallas as pl


def kernel(x, edge_index, eigenvectors, W1, b1, W2, b2):
    raise NotImplementedError("write your pallas kernel here")


# speedup vs baseline: 13.5933x; 13.5933x over previous
import functools

import jax
import jax.numpy as jnp
from jax import lax
from jax.experimental import pallas as pl
from jax.experimental.pallas import tpu as pltpu
from jax.experimental.pallas import tpu_sc as plsc

N = 10000
E = 320000
D_IN = 128
D_HID = 128
D_OUT = 64

NC = 2
NS = 16
EPT = E // (NC * NS)
C = 80
NCHUNK = EPT // C
STRIPE = 624
TAIL = N - NS * STRIPE

_MESH = plsc.VectorSubcoreMesh(core_axis_name="c", subcore_axis_name="s")


@functools.partial(
    pl.kernel,
    out_type=jax.ShapeDtypeStruct((NC, N, 1), jnp.float32),
    mesh=_MESH,
    scratch_types=[
        pltpu.VMEM((C,), jnp.int32),
        pltpu.VMEM((C, 1), jnp.float32),
        pltpu.VMEM_SHARED((N, 1), jnp.float32),
    ],
)
def _deg_kernel(dst_hbm, zeros_hbm, ones_hbm, out_hbm, idx_v, ones_v, acc):
    c = lax.axis_index("c")
    s = lax.axis_index("s")

    @pl.when(s == 0)
    def _init():
        pltpu.sync_copy(zeros_hbm, acc)

    pltpu.sync_copy(ones_hbm, ones_v)
    plsc.subcore_barrier()

    base0 = (c * NS + s) * EPT

    def body(i, carry):
        pltpu.sync_copy(dst_hbm.at[pl.ds(base0 + i * C, C)], idx_v)
        pltpu.sync_copy(ones_v, acc.at[idx_v], add=True)
        return carry

    lax.fori_loop(0, NCHUNK, body, 0)
    plsc.subcore_barrier()

    @pl.when(s == 0)
    def _out():
        pltpu.sync_copy(acc, out_hbm.at[c])


def _make_scatter(D):
    @functools.partial(
        pl.kernel,
        out_type=jax.ShapeDtypeStruct((NC, N, D), jnp.float32),
        mesh=_MESH,
        scratch_types=[
            pltpu.VMEM((C,), jnp.int32),
            pltpu.VMEM((C,), jnp.int32),
            pltpu.VMEM((C, D), jnp.float32),
            pltpu.VMEM_SHARED((N, D), jnp.float32),
            pltpu.SemaphoreType.DMA,
        ],
        compiler_params=pltpu.CompilerParams(use_tc_tiling_on_sc=False),
    )
    def _scatter(g_hbm, src_hbm, dst_hbm, zeros_hbm, out_hbm,
                 src_v, dst_v, rows_v, acc, sem):
        c = lax.axis_index("c")
        s = lax.axis_index("s")

        pltpu.sync_copy(zeros_hbm.at[pl.ds(s * STRIPE, STRIPE)],
                        acc.at[pl.ds(s * STRIPE, STRIPE)])

        @pl.when(s == 0)
        def _init_tail():
            pltpu.sync_copy(zeros_hbm.at[pl.ds(NS * STRIPE, TAIL)],
                            acc.at[pl.ds(NS * STRIPE, TAIL)])

        plsc.subcore_barrier()

        base0 = (c * NS + s) * EPT

        def body(i, carry):
            base = base0 + i * C
            pltpu.sync_copy(src_hbm.at[pl.ds(base, C)], src_v)
            pltpu.sync_copy(dst_hbm.at[pl.ds(base, C)], dst_v)
            pltpu.async_copy(g_hbm.at[src_v], rows_v, sem).wait()
            pltpu.sync_copy(rows_v, acc.at[dst_v], add=True)
            return carry

        lax.fori_loop(0, NCHUNK, body, 0)
        plsc.subcore_barrier()

        pltpu.sync_copy(acc.at[pl.ds(s * STRIPE, STRIPE)],
                        out_hbm.at[c].at[pl.ds(s * STRIPE, STRIPE)])

        @pl.when(s == 0)
        def _out_tail():
            pltpu.sync_copy(acc.at[pl.ds(NS * STRIPE, TAIL)],
                            out_hbm.at[c].at[pl.ds(NS * STRIPE, TAIL)])

    return _scatter


_scatter_hid = _make_scatter(D_HID)
_scatter_out = _make_scatter(D_OUT)


_BLK = 1000


def _tc1_body(dp_ref, x_ref, w_ref, g_ref, dinv_ref):
    deg = dp_ref[0, :, 0] + dp_ref[1, :, 0] + 1.0
    dinv = lax.rsqrt(deg)[:, None]
    h = jnp.dot(x_ref[...], w_ref[...], preferred_element_type=jnp.float32)
    g_ref[...] = h * dinv
    dinv_ref[...] = dinv


def _tc1(degp, x, W1):
    return pl.pallas_call(
        _tc1_body,
        grid=(N // _BLK,),
        in_specs=[
            pl.BlockSpec((2, _BLK, 1), lambda i: (0, i, 0)),
            pl.BlockSpec((_BLK, D_IN), lambda i: (i, 0)),
            pl.BlockSpec((D_IN, D_HID), lambda i: (0, 0)),
        ],
        out_specs=[
            pl.BlockSpec((_BLK, D_HID), lambda i: (i, 0)),
            pl.BlockSpec((_BLK, 1), lambda i: (i, 0)),
        ],
        out_shape=[
            jax.ShapeDtypeStruct((N, D_HID), jnp.float32),
            jax.ShapeDtypeStruct((N, 1), jnp.float32),
        ],
    )(degp, x, W1)


def _tc2_body(sp_ref, g_ref, dinv_ref, b1_ref, w2_ref, g2_ref):
    agg = (sp_ref[0] + sp_ref[1] + g_ref[...]) * dinv_ref[...] + b1_ref[...]
    h1 = jnp.maximum(agg, 0.0)
    h2 = jnp.dot(h1, w2_ref[...], preferred_element_type=jnp.float32)
    g2_ref[...] = h2 * dinv_ref[...]


def _tc2(sp, g, dinv, b1, W2):
    return pl.pallas_call(
        _tc2_body,
        grid=(N // _BLK,),
        in_specs=[
            pl.BlockSpec((2, _BLK, D_HID), lambda i: (0, i, 0)),
            pl.BlockSpec((_BLK, D_HID), lambda i: (i, 0)),
            pl.BlockSpec((_BLK, 1), lambda i: (i, 0)),
            pl.BlockSpec((1, D_HID), lambda i: (0, 0)),
            pl.BlockSpec((D_HID, D_OUT), lambda i: (0, 0)),
        ],
        out_specs=pl.BlockSpec((_BLK, D_OUT), lambda i: (i, 0)),
        out_shape=jax.ShapeDtypeStruct((N, D_OUT), jnp.float32),
    )(sp, g, dinv, b1, W2)


def _tc3_body(sp_ref, g2_ref, dinv_ref, b2_ref, o_ref):
    z = (sp_ref[0] + sp_ref[1] + g2_ref[...]) * dinv_ref[...] + b2_ref[...]
    m = jnp.max(z, axis=1, keepdims=True)
    lse = jnp.log(jnp.sum(jnp.exp(z - m), axis=1, keepdims=True)) + m
    o_ref[...] = z - lse


def _tc3(sp2, g2, dinv, b2):
    return pl.pallas_call(
        _tc3_body,
        grid=(N // _BLK,),
        in_specs=[
            pl.BlockSpec((2, _BLK, D_OUT), lambda i: (0, i, 0)),
            pl.BlockSpec((_BLK, D_OUT), lambda i: (i, 0)),
            pl.BlockSpec((_BLK, 1), lambda i: (i, 0)),
            pl.BlockSpec((1, D_OUT), lambda i: (0, 0)),
        ],
        out_specs=pl.BlockSpec((_BLK, D_OUT), lambda i: (i, 0)),
        out_shape=jax.ShapeDtypeStruct((N, D_OUT), jnp.float32),
    )(sp2, g2, dinv, b2)


def kernel(x, edge_index, eigenvectors, W1, b1, W2, b2):
    del eigenvectors
    src = edge_index[0].astype(jnp.int32)
    dst = edge_index[1].astype(jnp.int32)

    zeros1 = jnp.zeros((N, 1), jnp.float32)
    ones_c = jnp.ones((C, 1), jnp.float32)
    zeros_hid = jnp.zeros((N, D_HID), jnp.float32)
    zeros_out = jnp.zeros((N, D_OUT), jnp.float32)

    degp = _deg_kernel(dst, zeros1, ones_c)
    g, dinv = _tc1(degp, x, W1)
    sp = _scatter_hid(g, src, dst, zeros_hid)
    g2 = _tc2(sp, g, dinv, b1.reshape(1, D_HID), W2)
    sp2 = _scatter_out(g2, src, dst, zeros_out)
    return _tc3(sp2, g2, dinv, b2.reshape(1, D_OUT))


# speedup vs baseline: 32.3254x; 2.3780x over previous
import functools

import jax
import jax.numpy as jnp
from jax import lax
from jax.experimental import pallas as pl
from jax.experimental.pallas import tpu as pltpu
from jax.experimental.pallas import tpu_sc as plsc

N = 10000
E = 320000
D_IN = 128
D_HID = 128
D_OUT = 64

NC = 2
NS = 16
EPT = E // (NC * NS)
C = 100
NCHUNK = EPT // C
STRIPE = 624
TAIL = N - NS * STRIPE

_MESH = plsc.VectorSubcoreMesh(core_axis_name="c", subcore_axis_name="s")


@functools.partial(
    pl.kernel,
    out_type=jax.ShapeDtypeStruct((NC, N, 1), jnp.float32),
    mesh=_MESH,
    scratch_types=[
        pltpu.VMEM((EPT,), jnp.int32),
        pltpu.VMEM((EPT, 1), jnp.float32),
        pltpu.VMEM_SHARED((N, 1), jnp.float32),
    ],
    compiler_params=pltpu.CompilerParams(use_tc_tiling_on_sc=False),
)
def _deg_kernel(dst_hbm, zeros_hbm, ones_hbm, out_hbm, idx_v, ones_v, acc):
    c = lax.axis_index("c")
    s = lax.axis_index("s")

    @pl.when(s == 0)
    def _init():
        pltpu.sync_copy(zeros_hbm, acc)

    pltpu.sync_copy(ones_hbm, ones_v)
    pltpu.sync_copy(dst_hbm.at[pl.ds((c * NS + s) * EPT, EPT)], idx_v)
    plsc.subcore_barrier()

    pltpu.sync_copy(ones_v, acc.at[idx_v], add=True)
    plsc.subcore_barrier()

    @pl.when(s == 0)
    def _out():
        pltpu.sync_copy(acc, out_hbm.at[c])


def _make_scatter(D, NBUF):
    @functools.partial(
        pl.kernel,
        out_type=jax.ShapeDtypeStruct((NC, N, D), jnp.float32),
        mesh=_MESH,
        scratch_types=(
            [pltpu.VMEM((NCHUNK, C), jnp.int32)]
            + [pltpu.VMEM((C,), jnp.int32)] * NBUF
            + [pltpu.VMEM((C, D), jnp.float32)] * NBUF
            + [pltpu.VMEM_SHARED((N, D), jnp.float32)]
            + [pltpu.SemaphoreType.DMA] * (3 * NBUF)
        ),
        compiler_params=pltpu.CompilerParams(use_tc_tiling_on_sc=False),
    )
    def _scatter(g_hbm, src_hbm, dst_hbm, zeros_hbm, out_hbm,
                 src_i, *rest):
        c = lax.axis_index("c")
        s = lax.axis_index("s")
        droot = rest[:NBUF]
        bufs = rest[NBUF:2 * NBUF]
        acc = rest[2 * NBUF]
        gsem = rest[2 * NBUF + 1:3 * NBUF + 1]
        dsem = rest[3 * NBUF + 1:4 * NBUF + 1]
        ssem = rest[4 * NBUF + 1:]

        pltpu.sync_copy(zeros_hbm.at[pl.ds(s * STRIPE, STRIPE)],
                        acc.at[pl.ds(s * STRIPE, STRIPE)])

        @pl.when(s == 0)
        def _init_tail():
            pltpu.sync_copy(zeros_hbm.at[pl.ds(NS * STRIPE, TAIL)],
                            acc.at[pl.ds(NS * STRIPE, TAIL)])

        row0 = (c * NS + s) * NCHUNK
        pltpu.sync_copy(src_hbm.at[pl.ds(row0, NCHUNK)], src_i)
        plsc.subcore_barrier()

        for b in range(NBUF):
            pltpu.async_copy(dst_hbm.at[row0 + b], droot[b], dsem[b])
            pltpu.async_copy(g_hbm.at[src_i.at[b]], bufs[b], gsem[b])

        def body(j, carry):
            for b in range(NBUF):
                i = j * NBUF + b
                pltpu.make_async_copy(g_hbm.at[src_i.at[i]], bufs[b],
                                      gsem[b]).wait()
                pltpu.make_async_copy(dst_hbm.at[row0 + i], droot[b],
                                      dsem[b]).wait()
                pltpu.async_copy(bufs[b], acc.at[droot[b]], ssem[b],
                                 add=True)
                pltpu.make_async_copy(bufs[b], acc.at[droot[b]],
                                      ssem[b]).wait()

                @pl.when(i + NBUF < NCHUNK)
                def _next():
                    pltpu.async_copy(dst_hbm.at[row0 + i + NBUF], droot[b],
                                     dsem[b])
                    pltpu.async_copy(g_hbm.at[src_i.at[i + NBUF]], bufs[b],
                                     gsem[b])

            return carry

        lax.fori_loop(0, NCHUNK // NBUF, body, 0)
        plsc.subcore_barrier()

        pltpu.sync_copy(acc.at[pl.ds(s * STRIPE, STRIPE)],
                        out_hbm.at[c].at[pl.ds(s * STRIPE, STRIPE)])

        @pl.when(s == 0)
        def _out_tail():
            pltpu.sync_copy(acc.at[pl.ds(NS * STRIPE, TAIL)],
                            out_hbm.at[c].at[pl.ds(NS * STRIPE, TAIL)])

    return _scatter


_scatter_hid = _make_scatter(D_HID, 2)
_scatter_out = _make_scatter(D_OUT, 4)


_BLK = 1000


def _tc1_body(dp_ref, x_ref, w_ref, g_ref, dinv_ref):
    deg = dp_ref[0, :, 0] + dp_ref[1, :, 0] + 1.0
    dinv = lax.rsqrt(deg)[:, None]
    h = jnp.dot(x_ref[...], w_ref[...], preferred_element_type=jnp.float32)
    g_ref[...] = h * dinv
    dinv_ref[...] = dinv


def _tc1(degp, x, W1):
    return pl.pallas_call(
        _tc1_body,
        grid=(N // _BLK,),
        in_specs=[
            pl.BlockSpec((2, _BLK, 1), lambda i: (0, i, 0)),
            pl.BlockSpec((_BLK, D_IN), lambda i: (i, 0)),
            pl.BlockSpec((D_IN, D_HID), lambda i: (0, 0)),
        ],
        out_specs=[
            pl.BlockSpec((_BLK, D_HID), lambda i: (i, 0)),
            pl.BlockSpec((_BLK, 1), lambda i: (i, 0)),
        ],
        out_shape=[
            jax.ShapeDtypeStruct((N, D_HID), jnp.float32),
            jax.ShapeDtypeStruct((N, 1), jnp.float32),
        ],
    )(degp, x, W1)


def _tc2_body(sp_ref, g_ref, dinv_ref, b1_ref, w2_ref, g2_ref):
    agg = (sp_ref[0] + sp_ref[1] + g_ref[...]) * dinv_ref[...] + b1_ref[...]
    h1 = jnp.maximum(agg, 0.0)
    h2 = jnp.dot(h1, w2_ref[...], preferred_element_type=jnp.float32)
    g2_ref[...] = h2 * dinv_ref[...]


def _tc2(sp, g, dinv, b1, W2):
    return pl.pallas_call(
        _tc2_body,
        grid=(N // _BLK,),
        in_specs=[
            pl.BlockSpec((2, _BLK, D_HID), lambda i: (0, i, 0)),
            pl.BlockSpec((_BLK, D_HID), lambda i: (i, 0)),
            pl.BlockSpec((_BLK, 1), lambda i: (i, 0)),
            pl.BlockSpec((1, D_HID), lambda i: (0, 0)),
            pl.BlockSpec((D_HID, D_OUT), lambda i: (0, 0)),
        ],
        out_specs=pl.BlockSpec((_BLK, D_OUT), lambda i: (i, 0)),
        out_shape=jax.ShapeDtypeStruct((N, D_OUT), jnp.float32),
    )(sp, g, dinv, b1, W2)


def _tc3_body(sp_ref, g2_ref, dinv_ref, b2_ref, o_ref):
    z = (sp_ref[0] + sp_ref[1] + g2_ref[...]) * dinv_ref[...] + b2_ref[...]
    m = jnp.max(z, axis=1, keepdims=True)
    lse = jnp.log(jnp.sum(jnp.exp(z - m), axis=1, keepdims=True)) + m
    o_ref[...] = z - lse


def _tc3(sp2, g2, dinv, b2):
    return pl.pallas_call(
        _tc3_body,
        grid=(N // _BLK,),
        in_specs=[
            pl.BlockSpec((2, _BLK, D_OUT), lambda i: (0, i, 0)),
            pl.BlockSpec((_BLK, D_OUT), lambda i: (i, 0)),
            pl.BlockSpec((_BLK, 1), lambda i: (i, 0)),
            pl.BlockSpec((1, D_OUT), lambda i: (0, 0)),
        ],
        out_specs=pl.BlockSpec((_BLK, D_OUT), lambda i: (i, 0)),
        out_shape=jax.ShapeDtypeStruct((N, D_OUT), jnp.float32),
    )(sp2, g2, dinv, b2)


def kernel(x, edge_index, eigenvectors, W1, b1, W2, b2):
    del eigenvectors
    src = edge_index[0].astype(jnp.int32)
    dst = edge_index[1].astype(jnp.int32)
    src2d = src.reshape(E // C, C)
    dst2d = dst.reshape(E // C, C)

    zeros1 = jnp.zeros((N, 1), jnp.float32)
    ones_e = jnp.ones((EPT, 1), jnp.float32)
    zeros_hid = jnp.zeros((N, D_HID), jnp.float32)
    zeros_out = jnp.zeros((N, D_OUT), jnp.float32)

    degp = _deg_kernel(dst, zeros1, ones_e)
    g, dinv = _tc1(degp, x, W1)
    sp = _scatter_hid(g, src2d, dst2d, zeros_hid)
    g2 = _tc2(sp, g, dinv, b1.reshape(1, D_HID), W2)
    sp2 = _scatter_out(g2, src2d, dst2d, zeros_out)
    return _tc3(sp2, g2, dinv, b2.reshape(1, D_OUT))


# speedup vs baseline: 34.5748x; 1.0696x over previous
import functools

import jax
import jax.numpy as jnp
from jax import lax
from jax.experimental import pallas as pl
from jax.experimental.pallas import tpu as pltpu
from jax.experimental.pallas import tpu_sc as plsc

N = 10000
E = 320000
D_IN = 128
D_HID = 128
D_OUT = 64

NC = 2
NS = 16
EPT = E // (NC * NS)
STRIPE = 624
TAIL = N - NS * STRIPE

_MESH = plsc.VectorSubcoreMesh(core_axis_name="c", subcore_axis_name="s")


@functools.partial(
    pl.kernel,
    out_type=jax.ShapeDtypeStruct((NC, N, 1), jnp.float32),
    mesh=_MESH,
    scratch_types=[
        pltpu.VMEM((EPT,), jnp.int32),
        pltpu.VMEM((EPT, 1), jnp.float32),
        pltpu.VMEM_SHARED((N, 1), jnp.float32),
    ],
    compiler_params=pltpu.CompilerParams(use_tc_tiling_on_sc=False),
)
def _deg_kernel(dst_hbm, zeros_hbm, ones_hbm, out_hbm, idx_v, ones_v, acc):
    c = lax.axis_index("c")
    s = lax.axis_index("s")

    @pl.when(s == 0)
    def _init():
        pltpu.sync_copy(zeros_hbm, acc)

    pltpu.sync_copy(ones_hbm, ones_v)
    pltpu.sync_copy(dst_hbm.at[pl.ds((c * NS + s) * EPT, EPT)], idx_v)
    plsc.subcore_barrier()

    pltpu.sync_copy(ones_v, acc.at[idx_v], add=True)
    plsc.subcore_barrier()

    @pl.when(s == 0)
    def _out():
        pltpu.sync_copy(acc, out_hbm.at[c])


def _make_scatter(D, NBUF, C):
    NCHUNK = EPT // C

    @functools.partial(
        pl.kernel,
        out_type=jax.ShapeDtypeStruct((NC, N, D), jnp.float32),
        mesh=_MESH,
        scratch_types=(
            [pltpu.VMEM((NCHUNK, C), jnp.int32)]
            + [pltpu.VMEM((C,), jnp.int32)] * NBUF
            + [pltpu.VMEM((C, D), jnp.float32)] * NBUF
            + [pltpu.VMEM_SHARED((N, D), jnp.float32)]
            + [pltpu.SemaphoreType.DMA] * (3 * NBUF)
        ),
        compiler_params=pltpu.CompilerParams(use_tc_tiling_on_sc=False),
    )
    def _scatter(g_hbm, src_hbm, dst_hbm, zeros_hbm, out_hbm,
                 src_i, *rest):
        c = lax.axis_index("c")
        s = lax.axis_index("s")
        droot = rest[:NBUF]
        bufs = rest[NBUF:2 * NBUF]
        acc = rest[2 * NBUF]
        gsem = rest[2 * NBUF + 1:3 * NBUF + 1]
        dsem = rest[3 * NBUF + 1:4 * NBUF + 1]
        ssem = rest[4 * NBUF + 1:]

        pltpu.sync_copy(zeros_hbm.at[pl.ds(s * STRIPE, STRIPE)],
                        acc.at[pl.ds(s * STRIPE, STRIPE)])

        @pl.when(s == 0)
        def _init_tail():
            pltpu.sync_copy(zeros_hbm.at[pl.ds(NS * STRIPE, TAIL)],
                            acc.at[pl.ds(NS * STRIPE, TAIL)])

        row0 = (c * NS + s) * NCHUNK
        pltpu.sync_copy(src_hbm.at[pl.ds(row0, NCHUNK)], src_i)
        plsc.subcore_barrier()

        for b in range(NBUF):
            pltpu.async_copy(dst_hbm.at[row0 + b], droot[b], dsem[b])
            pltpu.async_copy(g_hbm.at[src_i.at[b]], bufs[b], gsem[b])

        def body(j, carry):
            for b in range(NBUF):
                i = j * NBUF + b
                pltpu.make_async_copy(g_hbm.at[src_i.at[i]], bufs[b],
                                      gsem[b]).wait()
                pltpu.make_async_copy(dst_hbm.at[row0 + i], droot[b],
                                      dsem[b]).wait()
                pltpu.async_copy(bufs[b], acc.at[droot[b]], ssem[b],
                                 add=True)
                pltpu.make_async_copy(bufs[b], acc.at[droot[b]],
                                      ssem[b]).wait()

                @pl.when(i + NBUF < NCHUNK)
                def _next():
                    pltpu.async_copy(dst_hbm.at[row0 + i + NBUF], droot[b],
                                     dsem[b])
                    pltpu.async_copy(g_hbm.at[src_i.at[i + NBUF]], bufs[b],
                                     gsem[b])

            return carry

        lax.fori_loop(0, NCHUNK // NBUF, body, 0)
        plsc.subcore_barrier()

        pltpu.sync_copy(acc.at[pl.ds(s * STRIPE, STRIPE)],
                        out_hbm.at[c].at[pl.ds(s * STRIPE, STRIPE)])

        @pl.when(s == 0)
        def _out_tail():
            pltpu.sync_copy(acc.at[pl.ds(NS * STRIPE, TAIL)],
                            out_hbm.at[c].at[pl.ds(NS * STRIPE, TAIL)])

    return _scatter


C_HID = 100
C_OUT = 100
assert (EPT // C_HID) % 2 == 0 and (EPT // C_OUT) % 4 == 0
_scatter_hid = _make_scatter(D_HID, 2, C_HID)
_scatter_out = _make_scatter(D_OUT, 4, C_OUT)


_BLK = 1000


def _tc1_body(dp_ref, x_ref, w_ref, g_ref, dinv_ref):
    deg = dp_ref[0, :, 0] + dp_ref[1, :, 0] + 1.0
    dinv = lax.rsqrt(deg)[:, None]
    h = jnp.dot(x_ref[...], w_ref[...], preferred_element_type=jnp.float32)
    g_ref[...] = h * dinv
    dinv_ref[...] = dinv


def _tc1(degp, x, W1):
    return pl.pallas_call(
        _tc1_body,
        grid=(N // _BLK,),
        in_specs=[
            pl.BlockSpec((2, _BLK, 1), lambda i: (0, i, 0)),
            pl.BlockSpec((_BLK, D_IN), lambda i: (i, 0)),
            pl.BlockSpec((D_IN, D_HID), lambda i: (0, 0)),
        ],
        out_specs=[
            pl.BlockSpec((_BLK, D_HID), lambda i: (i, 0)),
            pl.BlockSpec((_BLK, 1), lambda i: (i, 0)),
        ],
        out_shape=[
            jax.ShapeDtypeStruct((N, D_HID), jnp.float32),
            jax.ShapeDtypeStruct((N, 1), jnp.float32),
        ],
    )(degp, x, W1)


def _tc2_body(sp_ref, g_ref, dinv_ref, b1_ref, w2_ref, g2_ref):
    agg = (sp_ref[0] + sp_ref[1] + g_ref[...]) * dinv_ref[...] + b1_ref[...]
    h1 = jnp.maximum(agg, 0.0)
    h2 = jnp.dot(h1, w2_ref[...], preferred_element_type=jnp.float32)
    g2_ref[...] = h2 * dinv_ref[...]


def _tc2(sp, g, dinv, b1, W2):
    return pl.pallas_call(
        _tc2_body,
        grid=(N // _BLK,),
        in_specs=[
            pl.BlockSpec((2, _BLK, D_HID), lambda i: (0, i, 0)),
            pl.BlockSpec((_BLK, D_HID), lambda i: (i, 0)),
            pl.BlockSpec((_BLK, 1), lambda i: (i, 0)),
            pl.BlockSpec((1, D_HID), lambda i: (0, 0)),
            pl.BlockSpec((D_HID, D_OUT), lambda i: (0, 0)),
        ],
        out_specs=pl.BlockSpec((_BLK, D_OUT), lambda i: (i, 0)),
        out_shape=jax.ShapeDtypeStruct((N, D_OUT), jnp.float32),
    )(sp, g, dinv, b1, W2)


def _tc3_body(sp_ref, g2_ref, dinv_ref, b2_ref, o_ref):
    z = (sp_ref[0] + sp_ref[1] + g2_ref[...]) * dinv_ref[...] + b2_ref[...]
    m = jnp.max(z, axis=1, keepdims=True)
    lse = jnp.log(jnp.sum(jnp.exp(z - m), axis=1, keepdims=True)) + m
    o_ref[...] = z - lse


def _tc3(sp2, g2, dinv, b2):
    return pl.pallas_call(
        _tc3_body,
        grid=(N // _BLK,),
        in_specs=[
            pl.BlockSpec((2, _BLK, D_OUT), lambda i: (0, i, 0)),
            pl.BlockSpec((_BLK, D_OUT), lambda i: (i, 0)),
            pl.BlockSpec((_BLK, 1), lambda i: (i, 0)),
            pl.BlockSpec((1, D_OUT), lambda i: (0, 0)),
        ],
        out_specs=pl.BlockSpec((_BLK, D_OUT), lambda i: (i, 0)),
        out_shape=jax.ShapeDtypeStruct((N, D_OUT), jnp.float32),
    )(sp2, g2, dinv, b2)


def kernel(x, edge_index, eigenvectors, W1, b1, W2, b2):
    del eigenvectors
    src = edge_index[0].astype(jnp.int32)
    dst = edge_index[1].astype(jnp.int32)
    src_h = src.reshape(E // C_HID, C_HID)
    dst_h = dst.reshape(E // C_HID, C_HID)
    src_o = src.reshape(E // C_OUT, C_OUT)
    dst_o = dst.reshape(E // C_OUT, C_OUT)

    zeros1 = jnp.zeros((N, 1), jnp.float32)
    ones_e = jnp.ones((EPT, 1), jnp.float32)
    zeros_hid = jnp.zeros((N, D_HID), jnp.float32)
    zeros_out = jnp.zeros((N, D_OUT), jnp.float32)

    degp = _deg_kernel(dst, zeros1, ones_e)
    g, dinv = _tc1(degp, x, W1)
    sp = _scatter_hid(g, src_h, dst_h, zeros_hid)
    g2 = _tc2(sp, g, dinv, b1.reshape(1, D_HID), W2)
    sp2 = _scatter_out(g2, src_o, dst_o, zeros_out)
    return _tc3(sp2, g2, dinv, b2.reshape(1, D_OUT))
